# baseline (speedup 1.0000x reference)
import functools

import jax
import jax.numpy as jnp
from jax import lax
from jax.experimental import pallas as pl
from jax.experimental.pallas import tpu as pltpu
from jax.experimental.pallas import tpu_sc as plsc

NUM_FIELDS = 26
VOCAB = 100000
EMBED_DIM = 32
BATCH = 4096

NC, NS, L = 2, 16, 16
NW = NC * NS
NQ = NUM_FIELDS * EMBED_DIM
PER_W = NQ // NW


def _multi_embed(x_t, tbl2):
    mesh = plsc.VectorSubcoreMesh(core_axis_name="c", subcore_axis_name="s")

    @functools.partial(
        pl.kernel,
        mesh=mesh,
        out_type=jax.ShapeDtypeStruct((NQ, BATCH), jnp.float32),
        scratch_types=[
            pltpu.VMEM((VOCAB,), jnp.float32),
            pltpu.VMEM((BATCH,), jnp.int32),
            pltpu.VMEM((BATCH,), jnp.float32),
            pltpu.SemaphoreType.DMA,
            pltpu.SemaphoreType.DMA,
        ],
        compiler_params=pltpu.CompilerParams(
            use_tc_tiling_on_sc=True, needs_layout_passes=False
        ),
    )
    def k(xt_hbm, tbl_hbm, out_hbm, tblrow_v, idx_v, row_v, wsem, sem):
        wid = lax.axis_index("s") * NC + lax.axis_index("c")
        q0 = wid * PER_W

        def task(i, f_prev):
            q = q0 + i
            f = lax.div(q, jnp.int32(EMBED_DIM))
            cp_row = pltpu.async_copy(tbl_hbm.at[q], tblrow_v, sem)

            @pl.when(f != f_prev)
            def _():
                pltpu.sync_copy(xt_hbm.at[f], idx_v)

            cp_row.wait()

            @pl.when(i > 0)
            def _():
                pltpu.make_async_copy(row_v, out_hbm.at[q], wsem).wait()

            @plsc.parallel_loop(0, BATCH // L, unroll=16)
            def gath(j):
                sl = pl.ds(j * L, L)
                row_v[sl] = plsc.load_gather(tblrow_v, [idx_v[sl]])

            pltpu.async_copy(row_v, out_hbm.at[q], wsem)
            return f

        lax.fori_loop(0, PER_W, task, jnp.int32(-1))
        pltpu.make_async_copy(row_v, out_hbm.at[q0], wsem).wait()

    return k(x_t, tbl2)


def kernel(x, tables):
    tbl2 = tables.transpose(0, 2, 1).reshape(NQ, VOCAB)
    out_t = _multi_embed(x.T, tbl2)
    return out_t.T


# speedup vs baseline: 1.0044x; 1.0044x over previous
import functools

import jax
import jax.numpy as jnp
from jax import lax
from jax.experimental import pallas as pl
from jax.experimental.pallas import tpu as pltpu
from jax.experimental.pallas import tpu_sc as plsc

NUM_FIELDS = 26
VOCAB = 100000
EMBED_DIM = 32
BATCH = 4096

NC, NS, L = 2, 16, 16
NW = NC * NS
NQ = NUM_FIELDS * EMBED_DIM
PER_W = NQ // NW


def _multi_embed(x_t, tbl2):
    mesh = plsc.VectorSubcoreMesh(core_axis_name="c", subcore_axis_name="s")

    @functools.partial(
        pl.kernel,
        mesh=mesh,
        out_type=jax.ShapeDtypeStruct((NQ, BATCH), jnp.float32),
        scratch_types=[
            pltpu.VMEM((VOCAB,), jnp.float32),
            pltpu.VMEM((BATCH,), jnp.int32),
            pltpu.VMEM((BATCH,), jnp.float32),
            pltpu.SemaphoreType.DMA,
            pltpu.SemaphoreType.DMA,
        ],
        compiler_params=pltpu.CompilerParams(
            use_tc_tiling_on_sc=True, needs_layout_passes=False
        ),
    )
    def k(xt_hbm, tbl_hbm, out_hbm, tblrow_v, idx_v, row_v, wsem, sem):
        wid = lax.axis_index("s") * NC + lax.axis_index("c")
        q0 = wid * PER_W

        def task(i, f_prev):
            q = q0 + i
            f = lax.div(q, jnp.int32(EMBED_DIM))
            cp_row = pltpu.async_copy(tbl_hbm.at[q], tblrow_v, sem)

            @pl.when(f != f_prev)
            def _():
                pltpu.sync_copy(xt_hbm.at[f], idx_v)

            cp_row.wait()

            @pl.when(i > 0)
            def _():
                pltpu.make_async_copy(row_v, out_hbm.at[q], wsem).wait()

            @plsc.parallel_loop(0, BATCH // L, unroll=8)
            def gath(j):
                sl = pl.ds(j * L, L)
                row_v[sl] = plsc.load_gather(tblrow_v, [idx_v[sl]])

            pltpu.async_copy(row_v, out_hbm.at[q], wsem)
            return f

        lax.fori_loop(0, PER_W, task, jnp.int32(-1))
        pltpu.make_async_copy(row_v, out_hbm.at[q0], wsem).wait()

    return k(x_t, tbl2)


def kernel(x, tables):
    tbl2 = tables.transpose(0, 2, 1).reshape(NQ, VOCAB)
    out_t = _multi_embed(x.T, tbl2)
    return out_t.T
